# baseline (speedup 1.0000x reference)
import functools

import jax
import jax.numpy as jnp
from jax import lax
from jax.experimental import pallas as pl
from jax.experimental.pallas import tpu as pltpu
from jax.experimental.pallas import tpu_sc as plsc

_NUM_TABLES = 8
_NUM_EMB = 100000
_SHARD = 16
_PRIMES = (31, 43, 59, 61, 73, 97, 103, 113)
_BATCH = 4096
_SEQ = 200

_N_IDS = _BATCH * _SEQ
_NC, _NS, _L = 2, 16, 16
_NW = _NC * _NS
_ROWS_W = _BATCH // _NW
_RPC = 2
_C = _RPC * _SEQ
_GROWS = _C * _NUM_TABLES
_IDX_R = _GROWS // 128
_CHUNKS = _ROWS_W // _RPC

_mesh = plsc.VectorSubcoreMesh(
    core_axis_name="c", subcore_axis_name="s", num_cores=_NC, num_subcores=_NS
)


@functools.partial(
    pl.kernel,
    out_type=jax.ShapeDtypeStruct((_N_IDS, _NUM_TABLES, _SHARD), jnp.float32),
    mesh=_mesh,
    scratch_types=[
        pltpu.VMEM((_C,), jnp.int32),
        pltpu.VMEM((_C,), jnp.int32),
        pltpu.VMEM((_IDX_R, 128), jnp.int32),
        pltpu.VMEM((_IDX_R, 128), jnp.int32),
        pltpu.VMEM((_GROWS, _SHARD), jnp.float32),
        pltpu.VMEM((_GROWS, _SHARD), jnp.float32),
        pltpu.SemaphoreType.DMA,
        pltpu.SemaphoreType.DMA,
        pltpu.SemaphoreType.DMA,
        pltpu.SemaphoreType.DMA,
    ],
    compiler_params=pltpu.CompilerParams(use_tc_tiling_on_sc=False),
)
def _hash_embed_sc(ids_hbm, table_hbm, out_hbm,
                   ids_v0, ids_v1, idx_v0, idx_v1, rows_v0, rows_v1,
                   sem_g0, sem_g1, sem_o0, sem_o1):
    wid = lax.axis_index("s") * _NC + lax.axis_index("c")
    row_base = wid * _ROWS_W

    rcp = jnp.float32(1.0 / _NUM_EMB)
    ids_b = (ids_v0, ids_v1)
    idx_b = (idx_v0, idx_v1)
    rows_b = (rows_v0, rows_v1)
    gsems = (sem_g0, sem_g1)
    osems = (sem_o0, sem_o1)

    def stage_ids(c, b):
        r0 = row_base + c * _RPC
        for r in range(_RPC):
            pltpu.sync_copy(ids_hbm.at[r0 + r, :],
                            ids_b[b].at[pl.ds(r * _SEQ, _SEQ)])

    def compute_idx(b):
        ids_v, idx_v = ids_b[b], idx_b[b]

        def body(t, _):
            v = ids_v[pl.ds(t * _L, _L)] + 1
            for i in range(_NUM_TABLES):
                x = v * _PRIMES[i]
                q = (x.astype(jnp.float32) * rcp).astype(jnp.int32)
                r = x - q * _NUM_EMB
                r = jnp.where(r < 0, r + _NUM_EMB, r)
                r = jnp.where(r >= _NUM_EMB, r - _NUM_EMB, r)
                p = i * _C + t * _L
                idx_v[p >> 7, pl.ds(p & 127, _L)] = r + i * _NUM_EMB
            return 0
        lax.fori_loop(0, _C // _L, body, 0)

    def gather_descs(b):
        return [
            pltpu.make_async_copy(
                table_hbm.at[idx_b[b].at[j]],
                rows_b[b].at[pl.ds(j * 128, 128)],
                gsems[b],
            )
            for j in range(_IDX_R)
        ]

    def fire_gathers(b):
        for d in gather_descs(b):
            d.start()

    def wait_gathers(b):
        for d in gather_descs(b):
            d.wait()

    def out_descs(c, b):
        n0 = (row_base + c * _RPC) * _SEQ
        return [
            pltpu.make_async_copy(
                rows_b[b].at[pl.ds(i * _C, _C)],
                out_hbm.at[pl.ds(n0, _C), i],
                osems[b],
            )
            for i in range(_NUM_TABLES)
        ]

    def fire_out(c, b):
        for d in out_descs(c, b):
            d.start()

    def wait_out(c, b):
        for d in out_descs(c, b):
            d.wait()

    stage_ids(0, 0)
    compute_idx(0)
    fire_gathers(0)
    stage_ids(1, 1)
    compute_idx(1)
    fire_gathers(1)
    wait_gathers(0)
    fire_out(0, 0)

    def steady(k, _):
        c0 = 2 * k
        stage_ids(c0, 0)
        compute_idx(0)
        wait_out(c0 - 2, 0)
        fire_gathers(0)
        wait_gathers(1)
        fire_out(c0 - 1, 1)
        stage_ids(c0 + 1, 1)
        compute_idx(1)
        wait_out(c0 - 1, 1)
        fire_gathers(1)
        wait_gathers(0)
        fire_out(c0, 0)
        return 0

    lax.fori_loop(1, _CHUNKS // 2, steady, 0)

    wait_gathers(1)
    fire_out(_CHUNKS - 1, 1)
    wait_out(_CHUNKS - 2, 0)
    wait_out(_CHUNKS - 1, 1)


def kernel(input_ids, tables):
    table = tables.reshape(_NUM_TABLES * _NUM_EMB, _SHARD)
    out = _hash_embed_sc(input_ids, table)
    return out.reshape(_BATCH, _SEQ, _NUM_TABLES * _SHARD)


# speedup vs baseline: 1.0010x; 1.0010x over previous
import functools

import jax
import jax.numpy as jnp
from jax import lax
from jax.experimental import pallas as pl
from jax.experimental.pallas import tpu as pltpu
from jax.experimental.pallas import tpu_sc as plsc

_NUM_TABLES = 8
_NUM_EMB = 100000
_SHARD = 16
_PRIMES = (31, 43, 59, 61, 73, 97, 103, 113)
_BATCH = 4096
_SEQ = 200

_N_IDS = _BATCH * _SEQ
_NC, _NS, _L = 2, 16, 16
_NW = _NC * _NS
_ROWS_W = _BATCH // _NW
_RPC = 2
_C = _RPC * _SEQ
_GROWS = _C * _NUM_TABLES
_IDX_R = _GROWS // 128
_CHUNKS = _ROWS_W // _RPC

_mesh = plsc.VectorSubcoreMesh(
    core_axis_name="c", subcore_axis_name="s", num_cores=_NC, num_subcores=_NS
)


@functools.partial(
    pl.kernel,
    out_type=jax.ShapeDtypeStruct((_N_IDS, _NUM_TABLES, _SHARD), jnp.float32),
    mesh=_mesh,
    scratch_types=[
        pltpu.VMEM((_C,), jnp.int32),
        pltpu.VMEM((_C,), jnp.int32),
        pltpu.VMEM((_GROWS,), jnp.int32),
        pltpu.VMEM((_GROWS,), jnp.int32),
        pltpu.VMEM((_GROWS, _SHARD), jnp.float32),
        pltpu.VMEM((_GROWS, _SHARD), jnp.float32),
        pltpu.SemaphoreType.DMA,
        pltpu.SemaphoreType.DMA,
        pltpu.SemaphoreType.DMA,
        pltpu.SemaphoreType.DMA,
    ],
    compiler_params=pltpu.CompilerParams(use_tc_tiling_on_sc=False),
)
def _hash_embed_sc(ids_hbm, table3_hbm, out_hbm,
                   ids_v0, ids_v1, idx_v0, idx_v1, rows_v0, rows_v1,
                   sem_g0, sem_g1, sem_o0, sem_o1):
    wid = lax.axis_index("s") * _NC + lax.axis_index("c")
    row_base = wid * _ROWS_W

    rcp = jnp.float32(1.0 / _NUM_EMB)
    ids_b = (ids_v0, ids_v1)
    idx_b = (idx_v0, idx_v1)
    rows_b = (rows_v0, rows_v1)
    gsems = (sem_g0, sem_g1)
    osems = (sem_o0, sem_o1)

    def stage_ids(c, b):
        r0 = row_base + c * _RPC
        for r in range(_RPC):
            pltpu.sync_copy(ids_hbm.at[r0 + r, :],
                            ids_b[b].at[pl.ds(r * _SEQ, _SEQ)])

    def compute_idx(b):
        ids_v, idx_v = ids_b[b], idx_b[b]

        def body(t, _):
            v = ids_v[pl.ds(t * _L, _L)] + 1
            for i in range(_NUM_TABLES):
                x = v * _PRIMES[i]
                q = (x.astype(jnp.float32) * rcp).astype(jnp.int32)
                r = x - q * _NUM_EMB
                r = jnp.where(r < 0, r + _NUM_EMB, r)
                r = jnp.where(r >= _NUM_EMB, r - _NUM_EMB, r)
                idx_v[pl.ds(i * _C + t * _L, _L)] = r
            return 0
        lax.fori_loop(0, _C // _L, body, 0)

    _SPLITS = tuple(
        (u, 128 if u + 128 <= _C else _C - u) for u in range(0, _C, 128)
    )

    def gather_descs(b):
        return [
            pltpu.make_async_copy(
                table3_hbm.at[i].at[idx_b[b].at[pl.ds(i * _C + u, n)]],
                rows_b[b].at[pl.ds(i * _C + u, n)],
                gsems[b],
            )
            for i in range(_NUM_TABLES)
            for (u, n) in _SPLITS
        ]

    def fire_gathers(b):
        for d in gather_descs(b):
            d.start()

    def wait_gathers(b):
        for d in gather_descs(b):
            d.wait()

    def out_descs(c, b):
        n0 = (row_base + c * _RPC) * _SEQ
        return [
            pltpu.make_async_copy(
                rows_b[b].at[pl.ds(i * _C, _C)],
                out_hbm.at[pl.ds(n0, _C), i],
                osems[b],
            )
            for i in range(_NUM_TABLES)
        ]

    def fire_out(c, b):
        for d in out_descs(c, b):
            d.start()

    def wait_out(c, b):
        for d in out_descs(c, b):
            d.wait()

    stage_ids(0, 0)
    compute_idx(0)
    fire_gathers(0)
    stage_ids(1, 1)
    compute_idx(1)
    fire_gathers(1)
    wait_gathers(0)
    fire_out(0, 0)

    def steady(k, _):
        c0 = 2 * k
        stage_ids(c0, 0)
        compute_idx(0)
        wait_out(c0 - 2, 0)
        fire_gathers(0)
        wait_gathers(1)
        fire_out(c0 - 1, 1)
        stage_ids(c0 + 1, 1)
        compute_idx(1)
        wait_out(c0 - 1, 1)
        fire_gathers(1)
        wait_gathers(0)
        fire_out(c0, 0)
        return 0

    lax.fori_loop(1, _CHUNKS // 2, steady, 0)

    wait_gathers(1)
    fire_out(_CHUNKS - 1, 1)
    wait_out(_CHUNKS - 2, 0)
    wait_out(_CHUNKS - 1, 1)


def kernel(input_ids, tables):
    out = _hash_embed_sc(input_ids, tables)
    return out.reshape(_BATCH, _SEQ, _NUM_TABLES * _SHARD)


# speedup vs baseline: 1.1025x; 1.1014x over previous
import functools

import jax
import jax.numpy as jnp
from jax import lax
from jax.experimental import pallas as pl
from jax.experimental.pallas import tpu as pltpu
from jax.experimental.pallas import tpu_sc as plsc

_NUM_TABLES = 8
_NUM_EMB = 100000
_SHARD = 16
_PRIMES = (31, 43, 59, 61, 73, 97, 103, 113)
_BATCH = 4096
_SEQ = 200

_N_IDS = _BATCH * _SEQ
_NC, _NS, _L = 2, 16, 16
_NW = _NC * _NS
_ROWS_W = _BATCH // _NW
_RPC = 2
_C = _RPC * _SEQ
_GROWS = _C * _NUM_TABLES
_IDX_R = _GROWS // 128
_CHUNKS = _ROWS_W // _RPC

_mesh = plsc.VectorSubcoreMesh(
    core_axis_name="c", subcore_axis_name="s", num_cores=_NC, num_subcores=_NS
)


@functools.partial(
    pl.kernel,
    out_type=jax.ShapeDtypeStruct((_N_IDS * _NUM_TABLES, _SHARD), jnp.float32),
    mesh=_mesh,
    scratch_types=[
        pltpu.VMEM((2, _L), jnp.int32),
        pltpu.VMEM((_C,), jnp.int32),
        pltpu.VMEM((_C,), jnp.int32),
        pltpu.VMEM((_IDX_R, 128), jnp.int32),
        pltpu.VMEM((_IDX_R, 128), jnp.int32),
        pltpu.VMEM((_GROWS, _SHARD), jnp.float32),
        pltpu.VMEM((_GROWS, _SHARD), jnp.float32),
        pltpu.SemaphoreType.DMA,
        pltpu.SemaphoreType.DMA,
        pltpu.SemaphoreType.DMA,
        pltpu.SemaphoreType.DMA,
    ],
    compiler_params=pltpu.CompilerParams(use_tc_tiling_on_sc=False),
)
def _hash_embed_sc(ids_hbm, pv_hbm, table_hbm, out_hbm,
                   pv_v, ids_v0, ids_v1, idx_v0, idx_v1, rows_v0, rows_v1,
                   sem_g0, sem_g1, sem_o0, sem_o1):
    wid = lax.axis_index("s") * _NC + lax.axis_index("c")
    row_base = wid * _ROWS_W

    rcp = jnp.float32(1.0 / _NUM_EMB)
    pltpu.sync_copy(pv_hbm, pv_v)
    pvec = pv_v[0, :]
    ovec = pv_v[1, :]
    hilane = lax.iota(jnp.int32, _L) >= 8
    ids_b = (ids_v0, ids_v1)
    idx_b = (idx_v0, idx_v1)
    rows_b = (rows_v0, rows_v1)
    gsems = (sem_g0, sem_g1)
    osems = (sem_o0, sem_o1)

    def stage_ids(c, b):
        r0 = row_base + c * _RPC
        for r in range(_RPC):
            pltpu.sync_copy(ids_hbm.at[r0 + r, :],
                            ids_b[b].at[pl.ds(r * _SEQ, _SEQ)])

    def compute_idx(b):
        ids_v, idx_v = ids_b[b], idx_b[b]

        def body(t8, _):
            idsvec = ids_v[pl.ds(16 * t8, _L)]
            for u in range(8):
                a = idsvec[2 * u]
                b2 = idsvec[2 * u + 1]
                v = jnp.where(hilane, b2, a) + 1
                x = v * pvec
                q = (x.astype(jnp.float32) * rcp).astype(jnp.int32)
                r = x - q * _NUM_EMB
                r = jnp.where(r < 0, r + _NUM_EMB, r)
                r = jnp.where(r >= _NUM_EMB, r - _NUM_EMB, r)
                idx_v[t8, pl.ds(16 * u, _L)] = r + ovec
            return 0
        lax.fori_loop(0, _IDX_R, body, 0)

    def gather_descs(b):
        return [
            pltpu.make_async_copy(
                table_hbm.at[idx_b[b].at[j]],
                rows_b[b].at[pl.ds(j * 128, 128)],
                gsems[b],
            )
            for j in range(_IDX_R)
        ]

    def fire_gathers(b):
        for d in gather_descs(b):
            d.start()

    def wait_gathers(b):
        for d in gather_descs(b):
            d.wait()

    def out_desc(c, b):
        g0 = (row_base + c * _RPC) * _SEQ * _NUM_TABLES
        return pltpu.make_async_copy(
            rows_b[b],
            out_hbm.at[pl.ds(g0, _GROWS)],
            osems[b],
        )

    stage_ids(0, 0)
    compute_idx(0)
    fire_gathers(0)
    stage_ids(1, 1)
    compute_idx(1)
    fire_gathers(1)
    wait_gathers(0)
    out_desc(0, 0).start()

    def steady(k, _):
        c0 = 2 * k
        stage_ids(c0, 0)
        compute_idx(0)
        out_desc(c0 - 2, 0).wait()
        fire_gathers(0)
        wait_gathers(1)
        out_desc(c0 - 1, 1).start()
        stage_ids(c0 + 1, 1)
        compute_idx(1)
        out_desc(c0 - 1, 1).wait()
        fire_gathers(1)
        wait_gathers(0)
        out_desc(c0, 0).start()
        return 0

    lax.fori_loop(1, _CHUNKS // 2, steady, 0)

    wait_gathers(1)
    out_desc(_CHUNKS - 1, 1).start()
    out_desc(_CHUNKS - 2, 0).wait()
    out_desc(_CHUNKS - 1, 1).wait()


def kernel(input_ids, tables):
    table = tables.reshape(_NUM_TABLES * _NUM_EMB, _SHARD)
    pv = jnp.array(
        [list(_PRIMES) * 2,
         [i * _NUM_EMB for i in range(_NUM_TABLES)] * 2],
        dtype=jnp.int32,
    )
    out = _hash_embed_sc(input_ids, pv, table)
    return out.reshape(_BATCH, _SEQ, _NUM_TABLES * _SHARD)


# speedup vs baseline: 1.1133x; 1.0098x over previous
import functools

import jax
import jax.numpy as jnp
from jax import lax
from jax.experimental import pallas as pl
from jax.experimental.pallas import tpu as pltpu
from jax.experimental.pallas import tpu_sc as plsc

_NUM_TABLES = 8
_NUM_EMB = 100000
_SHARD = 16
_PRIMES = (31, 43, 59, 61, 73, 97, 103, 113)
_BATCH = 4096
_SEQ = 200

_N_IDS = _BATCH * _SEQ
_NC, _NS, _L = 2, 16, 16
_NW = _NC * _NS
_ROWS_W = _BATCH // _NW
_RPC = 2
_C = _RPC * _SEQ
_GROWS = _C * _NUM_TABLES
_IDX_R = _GROWS // 128
_CHUNKS = _ROWS_W // _RPC

_mesh = plsc.VectorSubcoreMesh(
    core_axis_name="c", subcore_axis_name="s", num_cores=_NC, num_subcores=_NS
)


@functools.partial(
    pl.kernel,
    out_type=jax.ShapeDtypeStruct((_N_IDS * _NUM_TABLES, _SHARD), jnp.float32),
    mesh=_mesh,
    scratch_types=[
        pltpu.VMEM((2, _L), jnp.int32),
        pltpu.VMEM((_C,), jnp.int32),
        pltpu.VMEM((_C,), jnp.int32),
        pltpu.VMEM((_IDX_R, 128), jnp.int32),
        pltpu.VMEM((_IDX_R, 128), jnp.int32),
        pltpu.VMEM((_GROWS, _SHARD), jnp.float32),
        pltpu.VMEM((_GROWS, _SHARD), jnp.float32),
        pltpu.SemaphoreType.DMA,
        pltpu.SemaphoreType.DMA,
        pltpu.SemaphoreType.DMA,
        pltpu.SemaphoreType.DMA,
    ],
    compiler_params=pltpu.CompilerParams(use_tc_tiling_on_sc=False),
)
def _hash_embed_sc(ids_hbm, pv_hbm, table_hbm, out_hbm,
                   pv_v, ids_v0, ids_v1, idx_v0, idx_v1, rows_v0, rows_v1,
                   sem_g0, sem_g1, sem_o0, sem_o1):
    wid = lax.axis_index("s") * _NC + lax.axis_index("c")
    row_base = wid * _ROWS_W

    rcp = jnp.float32(1.0 / _NUM_EMB)
    pltpu.sync_copy(pv_hbm, pv_v)
    pvec = pv_v[0, :]
    ovec = pv_v[1, :]
    hilane = lax.iota(jnp.int32, _L) >= 8
    ids_b = (ids_v0, ids_v1)
    idx_b = (idx_v0, idx_v1)
    rows_b = (rows_v0, rows_v1)
    gsems = (sem_g0, sem_g1)
    osems = (sem_o0, sem_o1)

    def stage_ids(c, b):
        r0 = row_base + c * _RPC
        for r in range(_RPC):
            pltpu.sync_copy(ids_hbm.at[r0 + r, :],
                            ids_b[b].at[pl.ds(r * _SEQ, _SEQ)])

    def compute_idx(b):
        ids_v, idx_v = ids_b[b], idx_b[b]

        def body(t8, _):
            idsvec = ids_v[pl.ds(16 * t8, _L)]
            for u in range(8):
                a = idsvec[2 * u]
                b2 = idsvec[2 * u + 1]
                v = jnp.where(hilane, b2, a) + 1
                x = v * pvec
                q = (x.astype(jnp.float32) * rcp).astype(jnp.int32)
                r = x - q * _NUM_EMB
                r = jnp.where(r < 0, r + _NUM_EMB, r)
                r = jnp.where(r >= _NUM_EMB, r - _NUM_EMB, r)
                idx_v[t8, pl.ds(16 * u, _L)] = (r << 3) + ovec
            return 0
        lax.fori_loop(0, _IDX_R, body, 0)

    def gather_descs(b):
        return [
            pltpu.make_async_copy(
                table_hbm.at[idx_b[b].at[j]],
                rows_b[b].at[pl.ds(j * 128, 128)],
                gsems[b],
            )
            for j in range(_IDX_R)
        ]

    def fire_gathers(b):
        for d in gather_descs(b):
            d.start()

    def wait_gathers(b):
        for d in gather_descs(b):
            d.wait()

    def out_desc(c, b):
        g0 = (row_base + c * _RPC) * _SEQ * _NUM_TABLES
        return pltpu.make_async_copy(
            rows_b[b],
            out_hbm.at[pl.ds(g0, _GROWS)],
            osems[b],
        )

    stage_ids(0, 0)
    compute_idx(0)
    fire_gathers(0)
    stage_ids(1, 1)
    compute_idx(1)
    fire_gathers(1)
    wait_gathers(0)
    out_desc(0, 0).start()

    def steady(k, _):
        c0 = 2 * k
        stage_ids(c0, 0)
        compute_idx(0)
        out_desc(c0 - 2, 0).wait()
        fire_gathers(0)
        wait_gathers(1)
        out_desc(c0 - 1, 1).start()
        stage_ids(c0 + 1, 1)
        compute_idx(1)
        out_desc(c0 - 1, 1).wait()
        fire_gathers(1)
        wait_gathers(0)
        out_desc(c0, 0).start()
        return 0

    lax.fori_loop(1, _CHUNKS // 2, steady, 0)

    wait_gathers(1)
    out_desc(_CHUNKS - 1, 1).start()
    out_desc(_CHUNKS - 2, 0).wait()
    out_desc(_CHUNKS - 1, 1).wait()


def kernel(input_ids, tables):
    table = tables.transpose(1, 0, 2).reshape(_NUM_TABLES * _NUM_EMB, _SHARD)
    pv = jnp.array(
        [list(_PRIMES) * 2,
         list(range(_NUM_TABLES)) * 2],
        dtype=jnp.int32,
    )
    out = _hash_embed_sc(input_ids, pv, table)
    return out.reshape(_BATCH, _SEQ, _NUM_TABLES * _SHARD)


# speedup vs baseline: 1.1201x; 1.0061x over previous
import functools

import jax
import jax.numpy as jnp
from jax import lax
from jax.experimental import pallas as pl
from jax.experimental.pallas import tpu as pltpu
from jax.experimental.pallas import tpu_sc as plsc

_NUM_TABLES = 8
_NUM_EMB = 100000
_SHARD = 16
_PRIMES = (31, 43, 59, 61, 73, 97, 103, 113)
_BATCH = 4096
_SEQ = 200

_N_IDS = _BATCH * _SEQ
_NC, _NS, _L = 2, 16, 16
_NW = _NC * _NS
_PER_W = _N_IDS // _NW
_C = 256
_GROWS = _C * _NUM_TABLES
_IDX_R = _GROWS // 128
_CHUNKS = _PER_W // _C
_NB = 3

_mesh = plsc.VectorSubcoreMesh(
    core_axis_name="c", subcore_axis_name="s", num_cores=_NC, num_subcores=_NS
)


@functools.partial(
    pl.kernel,
    out_type=jax.ShapeDtypeStruct((_N_IDS * _NUM_TABLES, _SHARD), jnp.float32),
    mesh=_mesh,
    scratch_types=(
        [pltpu.VMEM((2, _L), jnp.int32)]
        + [pltpu.VMEM((_C,), jnp.int32) for _ in range(_NB)]
        + [pltpu.VMEM((_IDX_R, 128), jnp.int32) for _ in range(_NB)]
        + [pltpu.VMEM((_GROWS, _SHARD), jnp.float32) for _ in range(_NB)]
        + [pltpu.SemaphoreType.DMA for _ in range(2 * _NB)]
    ),
    compiler_params=pltpu.CompilerParams(use_tc_tiling_on_sc=False),
)
def _hash_embed_sc(ids_hbm, pv_hbm, table_hbm, out_hbm, pv_v, *scr):
    ids_b = scr[0:_NB]
    idx_b = scr[_NB:2 * _NB]
    rows_b = scr[2 * _NB:3 * _NB]
    gsems = scr[3 * _NB:3 * _NB + _NB]
    osems = scr[3 * _NB + _NB:]

    wid = lax.axis_index("s") * _NC + lax.axis_index("c")
    base = wid * _PER_W

    rcp = jnp.float32(1.0 / _NUM_EMB)
    pltpu.sync_copy(pv_hbm, pv_v)
    pvec = pv_v[0, :]
    ovec = pv_v[1, :]
    hilane = lax.iota(jnp.int32, _L) >= 8

    def stage_ids(c, b):
        pltpu.sync_copy(ids_hbm.at[pl.ds(base + c * _C, _C)], ids_b[b])

    def compute_idx(b):
        ids_v, idx_v = ids_b[b], idx_b[b]

        def body(t8, _):
            idsvec = ids_v[pl.ds(16 * t8, _L)]
            for u in range(8):
                a = idsvec[2 * u]
                b2 = idsvec[2 * u + 1]
                v = jnp.where(hilane, b2, a) + 1
                x = v * pvec
                q = (x.astype(jnp.float32) * rcp).astype(jnp.int32)
                r = x - q * _NUM_EMB
                r = jnp.where(r < 0, r + _NUM_EMB, r)
                r = jnp.where(r >= _NUM_EMB, r - _NUM_EMB, r)
                idx_v[t8, pl.ds(16 * u, _L)] = (r << 3) + ovec
            return 0
        lax.fori_loop(0, _IDX_R, body, 0)

    def gather_descs(b):
        return [
            pltpu.make_async_copy(
                table_hbm.at[idx_b[b].at[j]],
                rows_b[b].at[pl.ds(j * 128, 128)],
                gsems[b],
            )
            for j in range(_IDX_R)
        ]

    def fire_gathers(b):
        for d in gather_descs(b):
            d.start()

    def wait_gathers(b):
        for d in gather_descs(b):
            d.wait()

    def out_desc(c, b):
        return pltpu.make_async_copy(
            rows_b[b],
            out_hbm.at[pl.ds((base + c * _C) * _NUM_TABLES, _GROWS)],
            osems[b],
        )

    stage_ids(0, 0)
    compute_idx(0)
    fire_gathers(0)
    stage_ids(1, 1)
    compute_idx(1)
    fire_gathers(1)
    wait_gathers(0)
    out_desc(0, 0).start()
    stage_ids(2, 2)
    compute_idx(2)
    fire_gathers(2)
    wait_gathers(1)
    out_desc(1, 1).start()

    def chunk_step(c, b):
        stage_ids(c, b)
        compute_idx(b)
        out_desc(c - _NB, b).wait()
        fire_gathers(b)
        wait_gathers((b + _NB - 1) % _NB)
        out_desc(c - 1, (b + _NB - 1) % _NB).start()

    def steady(k, _):
        c0 = _NB * k
        for d in range(_NB):
            chunk_step(c0 + d, d)
        return 0

    lax.fori_loop(1, (_CHUNKS - 1) // _NB, steady, 0)
    chunk_step(_CHUNKS - 1, (_CHUNKS - 1) % _NB)

    lb = (_CHUNKS - 1) % _NB
    out_desc(_CHUNKS - 3, (_CHUNKS - 3) % _NB).wait()
    out_desc(_CHUNKS - 2, (_CHUNKS - 2) % _NB).wait()
    wait_gathers(lb)
    out_desc(_CHUNKS - 1, lb).start()
    out_desc(_CHUNKS - 1, lb).wait()


def kernel(input_ids, tables):
    ids = input_ids.reshape(-1)
    table = tables.transpose(1, 0, 2).reshape(_NUM_TABLES * _NUM_EMB, _SHARD)
    pv = jnp.array(
        [list(_PRIMES) * 2,
         list(range(_NUM_TABLES)) * 2],
        dtype=jnp.int32,
    )
    out = _hash_embed_sc(ids, pv, table)
    return out.reshape(_BATCH, _SEQ, _NUM_TABLES * _SHARD)


# speedup vs baseline: 1.8384x; 1.6413x over previous
import functools

import jax
import jax.numpy as jnp
from jax import lax
from jax.experimental import pallas as pl
from jax.experimental.pallas import tpu as pltpu
from jax.experimental.pallas import tpu_sc as plsc

_NUM_TABLES = 8
_NUM_EMB = 100000
_SHARD = 16
_PRIMES = (31, 43, 59, 61, 73, 97, 103, 113)
_BATCH = 4096
_SEQ = 200

_N_IDS = _BATCH * _SEQ
_NC, _NS, _L = 2, 16, 16
_NW = _NC * _NS
_PER_W = _N_IDS // _NW
_C = 256
_GROWS = _C * _NUM_TABLES
_IDX_R = _GROWS // 128
_CHUNKS = _PER_W // _C
_NB = 3

_mesh = plsc.VectorSubcoreMesh(
    core_axis_name="c", subcore_axis_name="s", num_cores=_NC, num_subcores=_NS
)


@functools.partial(
    pl.kernel,
    out_type=jax.ShapeDtypeStruct((_N_IDS * _NUM_TABLES, _SHARD), jnp.float32),
    mesh=_mesh,
    scratch_types=(
        [pltpu.VMEM((2, _L), jnp.int32)]
        + [pltpu.VMEM((_C,), jnp.int32) for _ in range(_NB)]
        + [pltpu.VMEM((_IDX_R, 128), jnp.int32) for _ in range(_NB)]
        + [pltpu.VMEM((_GROWS, _SHARD), jnp.float32) for _ in range(_NB)]
        + [pltpu.SemaphoreType.DMA for _ in range(2 * _NB)]
    ),
    compiler_params=pltpu.CompilerParams(use_tc_tiling_on_sc=False),
)
def _hash_embed_sc(ids_hbm, pv_hbm, table_hbm, out_hbm, pv_v, *scr):
    ids_b = scr[0:_NB]
    idx_b = scr[_NB:2 * _NB]
    rows_b = scr[2 * _NB:3 * _NB]
    gsems = scr[3 * _NB:3 * _NB + _NB]
    osems = scr[3 * _NB + _NB:]

    wid = lax.axis_index("s") * _NC + lax.axis_index("c")
    base = wid * _PER_W

    rcp = jnp.float32(1.0 / _NUM_EMB)
    pltpu.sync_copy(pv_hbm, pv_v)
    pvec = pv_v[0, :]
    ovec = pv_v[1, :]
    hilane = lax.iota(jnp.int32, _L) >= 8

    def stage_ids(c, b):
        pltpu.sync_copy(ids_hbm.at[pl.ds(base + c * _C, _C)], ids_b[b])

    def compute_idx(b):
        ids_v, idx_v = ids_b[b], idx_b[b]

        def body(t8, _):
            idsvec = ids_v[pl.ds(16 * t8, _L)]
            for u in range(8):
                a = idsvec[2 * u]
                b2 = idsvec[2 * u + 1]
                v = jnp.where(hilane, b2, a) + 1
                x = v * pvec
                q = (x.astype(jnp.float32) * rcp).astype(jnp.int32)
                r = x - q * _NUM_EMB
                r = jnp.where(r < 0, r + _NUM_EMB, r)
                r = jnp.where(r >= _NUM_EMB, r - _NUM_EMB, r)
                idx_v[t8, pl.ds(16 * u, _L)] = (r << 3) + ovec
            return 0
        lax.fori_loop(0, _IDX_R, body, 0)

    def gather_descs(b):
        return [
            pltpu.make_async_copy(
                table_hbm.at[idx_b[b].at[j]],
                rows_b[b].at[pl.ds(j * 128, 128)],
                gsems[b],
            )
            for j in range(_IDX_R)
        ]

    def fire_gathers(b):
        for d in gather_descs(b):
            d.start()

    def wait_gathers(b):
        for d in gather_descs(b):
            d.wait()

    def out_desc(c, b):
        return pltpu.make_async_copy(
            rows_b[b],
            out_hbm.at[pl.ds((base + c * _C) * _NUM_TABLES, _GROWS)],
            osems[b],
        )

    stage_ids(0, 0)
    compute_idx(0)
    fire_gathers(0)
    stage_ids(1, 1)
    compute_idx(1)
    fire_gathers(1)
    wait_gathers(0)
    out_desc(0, 0).start()
    stage_ids(2, 2)
    compute_idx(2)
    fire_gathers(2)
    wait_gathers(1)
    out_desc(1, 1).start()

    def chunk_step(c, b):
        stage_ids(c, b)
        compute_idx(b)
        out_desc(c - _NB, b).wait()
        fire_gathers(b)
        wait_gathers((b + _NB - 1) % _NB)
        out_desc(c - 1, (b + _NB - 1) % _NB).start()

    def steady(k, _):
        c0 = _NB * k
        for d in range(_NB):
            chunk_step(c0 + d, d)
        return 0

    lax.fori_loop(1, (_CHUNKS - 1) // _NB, steady, 0)
    chunk_step(_CHUNKS - 1, (_CHUNKS - 1) % _NB)

    lb = (_CHUNKS - 1) % _NB
    out_desc(_CHUNKS - 3, (_CHUNKS - 3) % _NB).wait()
    out_desc(_CHUNKS - 2, (_CHUNKS - 2) % _NB).wait()
    wait_gathers(lb)
    out_desc(_CHUNKS - 1, lb).start()
    out_desc(_CHUNKS - 1, lb).wait()


def kernel(input_ids, tables):
    ids = input_ids.reshape(-1)
    t128 = lax.optimization_barrier(
        tables.transpose(1, 0, 2).reshape(_NUM_EMB, _NUM_TABLES * _SHARD))
    table = t128.reshape(_NUM_TABLES * _NUM_EMB, _SHARD)
    pv = jnp.array(
        [list(_PRIMES) * 2,
         list(range(_NUM_TABLES)) * 2],
        dtype=jnp.int32,
    )
    out = _hash_embed_sc(ids, pv, table)
    return out.reshape(_BATCH, _SEQ, _NUM_TABLES * _SHARD)


# speedup vs baseline: 1.8503x; 1.0065x over previous
import functools

import jax
import jax.numpy as jnp
from jax import lax
from jax.experimental import pallas as pl
from jax.experimental.pallas import tpu as pltpu
from jax.experimental.pallas import tpu_sc as plsc

_NUM_TABLES = 8
_NUM_EMB = 100000
_SHARD = 16
_PRIMES = (31, 43, 59, 61, 73, 97, 103, 113)
_BATCH = 4096
_SEQ = 200

_N_IDS = _BATCH * _SEQ
_NC, _NS, _L = 2, 16, 16
_NW = _NC * _NS
_PER_W = _N_IDS // _NW
_C = 256
_GROWS = _C * _NUM_TABLES
_IDX_R = _GROWS // 128
_CHUNKS = _PER_W // _C
_NB = 3

_mesh = plsc.VectorSubcoreMesh(
    core_axis_name="c", subcore_axis_name="s", num_cores=_NC, num_subcores=_NS
)


@functools.partial(
    pl.kernel,
    out_type=jax.ShapeDtypeStruct((_N_IDS * _NUM_TABLES, _SHARD), jnp.float32),
    mesh=_mesh,
    scratch_types=(
        [pltpu.VMEM((2, _L), jnp.int32)]
        + [pltpu.VMEM((_C,), jnp.int32) for _ in range(_NB)]
        + [pltpu.VMEM((_IDX_R, 128), jnp.int32) for _ in range(_NB)]
        + [pltpu.VMEM((_GROWS, _SHARD), jnp.float32) for _ in range(_NB)]
        + [pltpu.SemaphoreType.DMA for _ in range(3 * _NB)]
    ),
    compiler_params=pltpu.CompilerParams(use_tc_tiling_on_sc=False),
)
def _hash_embed_sc(ids_hbm, pv_hbm, table_hbm, out_hbm, pv_v, *scr):
    ids_b = scr[0:_NB]
    idx_b = scr[_NB:2 * _NB]
    rows_b = scr[2 * _NB:3 * _NB]
    gsems = scr[3 * _NB:4 * _NB]
    osems = scr[4 * _NB:5 * _NB]
    isems = scr[5 * _NB:6 * _NB]

    wid = lax.axis_index("s") * _NC + lax.axis_index("c")
    base = wid * _PER_W

    rcp = jnp.float32(1.0 / _NUM_EMB)
    pltpu.sync_copy(pv_hbm, pv_v)
    pvec = pv_v[0, :]
    ovec = pv_v[1, :]
    hilane = lax.iota(jnp.int32, _L) >= 8

    def ids_desc(c, b):
        return pltpu.make_async_copy(
            ids_hbm.at[pl.ds(base + c * _C, _C)], ids_b[b], isems[b])

    def compute_idx(b):
        ids_v, idx_v = ids_b[b], idx_b[b]

        def body(t8, _):
            idsvec = ids_v[pl.ds(16 * t8, _L)]
            for u in range(8):
                a = idsvec[2 * u]
                b2 = idsvec[2 * u + 1]
                v = jnp.where(hilane, b2, a) + 1
                x = v * pvec
                q = (x.astype(jnp.float32) * rcp).astype(jnp.int32)
                r = x - q * _NUM_EMB
                r = jnp.where(r < 0, r + _NUM_EMB, r)
                r = jnp.where(r >= _NUM_EMB, r - _NUM_EMB, r)
                idx_v[t8, pl.ds(16 * u, _L)] = (r << 3) + ovec
            return 0
        lax.fori_loop(0, _IDX_R, body, 0)

    def gather_descs(b):
        return [
            pltpu.make_async_copy(
                table_hbm.at[idx_b[b].at[j]],
                rows_b[b].at[pl.ds(j * 128, 128)],
                gsems[b],
            )
            for j in range(_IDX_R)
        ]

    def fire_gathers(b):
        for d in gather_descs(b):
            d.start()

    def wait_gathers(b):
        pltpu.make_async_copy(
            table_hbm.at[pl.ds(0, _GROWS)], rows_b[b], gsems[b]).wait()

    def out_desc(c, b):
        return pltpu.make_async_copy(
            rows_b[b],
            out_hbm.at[pl.ds((base + c * _C) * _NUM_TABLES, _GROWS)],
            osems[b],
        )

    for b in range(_NB):
        ids_desc(b, b).start()
    ids_desc(0, 0).wait()
    compute_idx(0)
    ids_desc(_NB, 0).start()
    fire_gathers(0)
    ids_desc(1, 1).wait()
    compute_idx(1)
    ids_desc(_NB + 1, 1).start()
    fire_gathers(1)
    wait_gathers(0)
    out_desc(0, 0).start()
    ids_desc(2, 2).wait()
    compute_idx(2)
    ids_desc(_NB + 2, 2).start()
    fire_gathers(2)
    wait_gathers(1)
    out_desc(1, 1).start()

    def chunk_step(c, b, prefetch):
        ids_desc(c, b).wait()
        compute_idx(b)
        if prefetch:
            @pl.when(c + _NB < _CHUNKS)
            def _():
                ids_desc(c + _NB, b).start()
        out_desc(c - _NB, b).wait()
        fire_gathers(b)
        wait_gathers((b + _NB - 1) % _NB)
        out_desc(c - 1, (b + _NB - 1) % _NB).start()

    def steady(k, _):
        c0 = _NB * k
        for d in range(_NB):
            chunk_step(c0 + d, d, True)
        return 0

    lax.fori_loop(1, (_CHUNKS - 1) // _NB, steady, 0)
    chunk_step(_CHUNKS - 1, (_CHUNKS - 1) % _NB, False)

    lb = (_CHUNKS - 1) % _NB
    out_desc(_CHUNKS - 3, (_CHUNKS - 3) % _NB).wait()
    out_desc(_CHUNKS - 2, (_CHUNKS - 2) % _NB).wait()
    wait_gathers(lb)
    out_desc(_CHUNKS - 1, lb).start()
    out_desc(_CHUNKS - 1, lb).wait()


def kernel(input_ids, tables):
    ids = input_ids.reshape(-1)
    t128 = lax.optimization_barrier(
        tables.transpose(1, 0, 2).reshape(_NUM_EMB, _NUM_TABLES * _SHARD))
    table = t128.reshape(_NUM_TABLES * _NUM_EMB, _SHARD)
    pv = jnp.array(
        [list(_PRIMES) * 2,
         list(range(_NUM_TABLES)) * 2],
        dtype=jnp.int32,
    )
    out = _hash_embed_sc(ids, pv, table)
    return out.reshape(_BATCH, _SEQ, _NUM_TABLES * _SHARD)
